# baseline (speedup 1.0000x reference)
import functools

import jax
import jax.numpy as jnp
from jax import lax
from jax.experimental import pallas as pl
from jax.experimental.pallas import tpu as pltpu
from jax.experimental.pallas import tpu_sc as plsc

_NC = 2
_NS = 16
_L = 16
_NW = _NC * _NS

_N = 10000
_D = 128
_E = 320000
_EPW = _E // _NW
_CH = 80
_NCH = _EPW // _CH
_RPS = _N // _NS
_ZR = 125
_HR = _N // _L

_mesh = plsc.VectorSubcoreMesh(core_axis_name="c", subcore_axis_name="s")
_sc_params = pltpu.CompilerParams(needs_layout_passes=False)


@functools.partial(
    pl.kernel,
    out_type=jax.ShapeDtypeStruct((_NC, _HR, _L), jnp.float32),
    mesh=_mesh,
    scratch_types=[
        pltpu.VMEM((_EPW,), jnp.int32),
        pltpu.VMEM((_HR, _L), jnp.float32),
        pltpu.VMEM((5, _ZR), jnp.int32),
        pltpu.VMEM_SHARED((_HR, _L), jnp.float32),
    ],
    compiler_params=_sc_params,
)
def _sc_degree_hist(dst_hbm, rix_hbm, out_hbm, idx_v, hist_v, rix_v, hist_sh):
    c = lax.axis_index("c")
    s = lax.axis_index("s")
    wid = s * _NC + c
    pltpu.sync_copy(dst_hbm.at[wid], idx_v)
    pltpu.sync_copy(rix_hbm, rix_v)
    z16 = jnp.zeros((_L,), jnp.float32)

    @pl.loop(0, _HR)
    def _zero(r):
        hist_v[r, :] = z16

    @pl.when(s == 0)
    def _():
        pltpu.sync_copy(hist_v, hist_sh)

    plsc.subcore_barrier()

    ones = jnp.ones((_L,), jnp.float32)
    four = jnp.full((_L,), 4, jnp.int32)
    fifteen = jnp.full((_L,), 15, jnp.int32)

    @pl.loop(0, _EPW, step=_L)
    def _hist(i):
        idx = idx_v[pl.ds(i, _L)]
        row = jnp.right_shift(idx, four)
        colv = jnp.bitwise_and(idx, fifteen)
        plsc.addupdate_scatter(hist_v, [row, colv], ones)

    @pl.loop(0, 5)
    def _combine(k):
        pltpu.sync_copy(hist_v.at[pl.ds(k * _ZR, _ZR)],
                        hist_sh.at[rix_v.at[k]], add=True)

    plsc.subcore_barrier()

    @pl.when(s == 0)
    def _():
        pltpu.sync_copy(hist_sh, out_hbm.at[c])


_HALF = _N // _NC
_TRASH = 64
_AROWS = 5120
_CAP = 126
_PAD = 160


@functools.partial(
    pl.kernel,
    out_type=[
        jax.ShapeDtypeStruct((_NW, 2, _CAP, _CH), jnp.int32),
        jax.ShapeDtypeStruct((_NW, 2, _CAP, _CH), jnp.int32),
        jax.ShapeDtypeStruct((_NW, 2, 8, _L), jnp.int32),
    ],
    mesh=_mesh,
    scratch_types=[
        pltpu.VMEM((_EPW,), jnp.int32),
        pltpu.VMEM((_EPW,), jnp.int32),
        pltpu.VMEM((_CAP, _CH), jnp.int32),
        pltpu.VMEM((_CAP, _CH), jnp.int32),
        pltpu.VMEM((_CAP, _CH), jnp.int32),
        pltpu.VMEM((_CAP, _CH), jnp.int32),
        pltpu.VMEM((8, _L), jnp.int32),
    ],
    compiler_params=_sc_params,
)
def _sc_partition(src_hbm, dst_hbm, psrc_hbm, pdst_hbm, pcnt_hbm,
                  sv, dv, cs0, cd0, cs1, cd1, cnt_v):
    c = lax.axis_index("c")
    s = lax.axis_index("s")
    wid = s * _NC + c
    pltpu.sync_copy(src_hbm.at[wid], sv)
    pltpu.sync_copy(dst_hbm.at[wid], dv)
    iota = lax.iota(jnp.int32, _L)
    chw = jnp.full((_L,), _CH, jnp.int32)
    halfv = jnp.full((_L,), _HALF, jnp.int32)

    def vec_body(v, carry):
        b0, b1 = carry
        s16 = sv[pl.ds(v * _L, _L)]
        d16 = dv[pl.ds(v * _L, _L)]
        m0 = d16 < halfv
        m1 = jnp.logical_not(m0)
        cum0 = plsc.cumsum(m0.astype(jnp.int32))
        cum1 = plsc.cumsum(m1.astype(jnp.int32))
        pos0 = cum0 + (b0 - 1)
        pos1 = cum1 + (b1 - 1)
        r0 = lax.div(pos0, chw)
        c0 = pos0 - r0 * chw
        r1 = lax.div(pos1, chw)
        c1 = pos1 - r1 * chw
        plsc.store_scatter(cs0, [r0, c0], s16, mask=m0)
        plsc.store_scatter(cd0, [r0, c0], d16, mask=m0)
        plsc.store_scatter(cs1, [r1, c1], s16, mask=m1)
        plsc.store_scatter(cd1, [r1, c1], d16 - halfv, mask=m1)
        return (b0 + jnp.sum(m0.astype(jnp.int32)),
                b1 + jnp.sum(m1.astype(jnp.int32)))

    b0, b1 = lax.fori_loop(0, _EPW // _L, vec_body,
                           (jnp.int32(0), jnp.int32(0)))

    zsrc = jnp.zeros((_L,), jnp.int32)
    trash = halfv + iota
    for b, csX, cdX, cnt in ((0, cs0, cd0, b0), (1, cs1, cd1, b1)):
        rounded = ((cnt + (_PAD - 1)) // _PAD) * _PAD
        for k in range(_PAD // _L):
            idx = cnt + k * _L + iota
            mask = idx < rounded
            rr = lax.div(idx, chw)
            cc = idx - rr * chw
            plsc.store_scatter(csX, [rr, cc], zsrc, mask=mask)
            plsc.store_scatter(cdX, [rr, cc], trash, mask=mask)
        n2 = rounded // _PAD
        n2v = jnp.zeros((_L,), jnp.int32) + n2

        @pl.loop(0, 8)
        def _fill(rrow):
            cnt_v[rrow, :] = n2v

        pltpu.sync_copy(csX, psrc_hbm.at[wid, b])
        pltpu.sync_copy(cdX, pdst_hbm.at[wid, b])
        pltpu.sync_copy(cnt_v, pcnt_hbm.at[wid, b])


@functools.partial(
    pl.kernel,
    out_type=jax.ShapeDtypeStruct((_NC, _HALF, _D), jnp.float32),
    mesh=_mesh,
    scratch_types=[
        pltpu.VMEM((_CAP, _CH), jnp.int32),
        pltpu.VMEM((_CAP, _CH), jnp.int32),
        pltpu.VMEM((8, _L), jnp.int32),
        pltpu.VMEM((_CH, _D), jnp.float32),
        pltpu.VMEM((_CH, _D), jnp.float32),
        pltpu.VMEM_SHARED((_AROWS, _D), jnp.float32),
        pltpu.SemaphoreType.DMA,
        pltpu.SemaphoreType.DMA,
    ],
    compiler_params=_sc_params,
)
def _sc_edge_agg(hs_hbm, psrc_hbm, pdst_hbm, pcnt_hbm, out_hbm,
                 src_v, dst_v, cbuf, rows_a, rows_b, agg_sh, gsem_a, gsem_b):
    c = lax.axis_index("c")
    s = lax.axis_index("s")
    z16 = jnp.zeros((_L,), jnp.float32)

    @pl.loop(0, _CH)
    def _zero(r):
        @pl.loop(0, _D, step=_L)
        def _(j):
            rows_a[r, pl.ds(j, _L)] = z16

    @pl.loop(0, 320, step=_CH)
    def _zero_spmem(r0):
        pltpu.sync_copy(rows_a, agg_sh.at[pl.ds(s * 320 + r0, _CH)])

    plsc.subcore_barrier()

    for seg in range(2):
        w = s * 2 + seg
        pltpu.sync_copy(psrc_hbm.at[w, c], src_v)
        pltpu.sync_copy(pdst_hbm.at[w, c], dst_v)
        pltpu.sync_copy(pcnt_hbm.at[w, c], cbuf)
        n2 = jnp.max(cbuf[0, :])

        @pl.when(n2 > 0)
        def _consume(n2=n2):
            pltpu.async_copy(hs_hbm.at[src_v.at[0]], rows_a, gsem_a)

            def body2(j2, carry):
                j = j2 * 2
                pltpu.async_copy(hs_hbm.at[src_v.at[j + 1]], rows_b, gsem_b)
                pltpu.make_async_copy(hs_hbm.at[src_v.at[j]], rows_a,
                                      gsem_a).wait()
                pltpu.sync_copy(rows_a, agg_sh.at[dst_v.at[j]], add=True)

                @pl.when(j2 + 1 < n2)
                def _():
                    pltpu.async_copy(hs_hbm.at[src_v.at[j + 2]], rows_a,
                                     gsem_a)

                pltpu.make_async_copy(hs_hbm.at[src_v.at[j + 1]], rows_b,
                                      gsem_b).wait()
                pltpu.sync_copy(rows_b, agg_sh.at[dst_v.at[j + 1]], add=True)
                return carry

            lax.fori_loop(0, n2, body2, jnp.int32(0))

    plsc.subcore_barrier()

    @pl.when(s < 5)
    def _writeout():
        pltpu.sync_copy(agg_sh.at[pl.ds(s * 1000, 1000)],
                        out_hbm.at[c, pl.ds(s * 1000, 1000)])


def _mm_body(x_ref, w_ref, o_ref):
    o_ref[...] = jnp.dot(x_ref[...], w_ref[...],
                         preferred_element_type=jnp.float32)


def _scale_body(h_ref, deg_ref, o_ref):
    o_ref[...] = h_ref[...] * lax.rsqrt(deg_ref[...])


def _mid_body(agg_ref, hs_ref, deg_ref, b1_ref, gamma_ref, beta_ref,
              w2_ref, o_ref):
    dinv = lax.rsqrt(deg_ref[...])
    t = (agg_ref[...] + hs_ref[...]) * dinv + b1_ref[...]
    mu = jnp.mean(t, axis=0, keepdims=True)
    var = jnp.mean(jnp.square(t - mu), axis=0, keepdims=True)
    hn = (t - mu) * lax.rsqrt(var + 1e-5) * gamma_ref[...] + beta_ref[...]
    h = jnp.maximum(hn, 0.0)
    o_ref[...] = jnp.dot(h, w2_ref[...],
                         preferred_element_type=jnp.float32) * dinv


def _fin_body(agg_ref, hs_ref, deg_ref, b2_ref, o_ref):
    o_ref[...] = ((agg_ref[...] + hs_ref[...]) * lax.rsqrt(deg_ref[...])
                  + b2_ref[...])


def kernel(x, edge_index, W1, b1, gamma, beta, W2, b2):
    n, d = x.shape
    src2 = edge_index[0].astype(jnp.int32).reshape(_NW, _EPW)
    dst2 = edge_index[1].astype(jnp.int32).reshape(_NW, _EPW)
    rix = jnp.arange(5 * _ZR, dtype=jnp.int32).reshape(5, _ZR)
    nd = jax.ShapeDtypeStruct((n, d), jnp.float32)

    psrc, pdst, pcnt = _sc_partition(src2, dst2)
    hist = _sc_degree_hist(dst2, rix)
    h1 = pl.pallas_call(_mm_body, out_shape=nd)(x, W1)
    deg = (hist[0] + hist[1] + 1.0).reshape(n, 1)
    hs1 = pl.pallas_call(_scale_body, out_shape=nd)(h1, deg)
    agg1 = _sc_edge_agg(hs1, psrc, pdst, pcnt).reshape(n, d)
    hs2 = pl.pallas_call(
        _mid_body,
        out_shape=nd,
    )(agg1, hs1, deg, b1.reshape(1, d), gamma.reshape(1, d),
      beta.reshape(1, d), W2)
    agg2 = _sc_edge_agg(hs2, psrc, pdst, pcnt).reshape(n, d)
    out = pl.pallas_call(
        _fin_body,
        out_shape=nd,
    )(agg2, hs2, deg, b2.reshape(1, d))
    return out
